# baseline (speedup 1.0000x reference)
import functools

import jax
import jax.numpy as jnp
from jax import lax
from jax.experimental import pallas as pl
from jax.experimental.pallas import tpu as pltpu
from jax.experimental.pallas import tpu_sc as plsc

NC = 2
NS = 16
NW = NC * NS

_EB_SC = 128


def _leaky(v):
    return jnp.where(v >= 0, v, 0.01 * v)


def _gather_pipelined(h_hbm, out_hbm, idx2d, ib0, ib1, bb, bn, rb0, rb1,
                      gs0, gs1, ss0, ss1, is0, is1):
    def i_start(j, ib, sem):
        pltpu.async_copy(idx2d.at[bb + j], ib, sem)

    def i_wait(ib, sem):
        pltpu.make_async_copy(idx2d.at[0], ib, sem).wait()

    def g_start(ib, rb, sem):
        pltpu.async_copy(h_hbm.at[ib], rb, sem)

    def g_wait(ib, rb, sem):
        pltpu.make_async_copy(h_hbm.at[ib], rb, sem).wait()

    def s_start(j, rb, sem):
        pltpu.async_copy(rb, out_hbm.at[pl.ds((bb + j) * _EB_SC, _EB_SC)],
                         sem)

    def s_wait(rb, sem):
        pltpu.make_async_copy(rb, out_hbm.at[pl.ds(0, _EB_SC)], sem).wait()

    i_start(0, ib0, is0)

    def pair(p, _):
        j0 = 2 * p
        j1 = j0 + 1
        i_start(j1, ib1, is1)
        i_wait(ib0, is0)
        g_start(ib0, rb0, gs0)
        g_wait(ib0, rb0, gs0)
        s_start(j0, rb0, ss0)
        i_wait(ib1, is1)
        g_start(ib1, rb1, gs1)

        @pl.when(j0 + 2 < bn)
        def _():
            i_start(j0 + 2, ib0, is0)

        g_wait(ib1, rb1, gs1)
        s_wait(rb0, ss0)
        s_start(j1, rb1, ss1)
        s_wait(rb1, ss1)
        return 0

    lax.fori_loop(0, bn // 2, pair, 0)


def _pack_bf16(x):
    n, w = x.shape
    return lax.bitcast_convert_type(x.reshape(n, w // 2, 2), jnp.float32)


def _unpack_bf16(x):
    n, k = x.shape
    return lax.bitcast_convert_type(x, jnp.bfloat16).reshape(n, 2 * k)


@functools.lru_cache(maxsize=None)
def _sc_gather2(E, N, W):
    bn = E // _EB_SC // NW
    assert bn % 2 == 0
    mesh = plsc.VectorSubcoreMesh(core_axis_name="c", subcore_axis_name="s",
                                  num_cores=NC, num_subcores=NS)
    out_type = [jax.ShapeDtypeStruct((E, W), jnp.float32),
                jax.ShapeDtypeStruct((E, W), jnp.float32)]
    scratch = [pltpu.VMEM((_EB_SC,), jnp.int32),
               pltpu.VMEM((_EB_SC,), jnp.int32),
               pltpu.VMEM((_EB_SC, W), jnp.float32),
               pltpu.VMEM((_EB_SC, W), jnp.float32)] + \
              [pltpu.SemaphoreType.DMA] * 6

    @functools.partial(pl.kernel, out_type=out_type, mesh=mesh,
                       scratch_types=scratch)
    def k(h_hbm, src2d, dst2d, hs_hbm, hd_hbm, ib0, ib1, rb0, rb1, gs0, gs1,
          ss0, ss1, is0, is1):
        wid = lax.axis_index("s") * NC + lax.axis_index("c")
        bb = wid * bn
        for idx2d, out_hbm in ((src2d, hs_hbm), (dst2d, hd_hbm)):
            _gather_pipelined(h_hbm, out_hbm, idx2d, ib0, ib1, bb, bn, rb0,
                              rb1, gs0, gs1, ss0, ss1, is0, is1)

    return k


@functools.lru_cache(maxsize=None)
def _sc_gather1(E, N, W):
    bn = E // _EB_SC // NW
    assert bn % 2 == 0
    mesh = plsc.VectorSubcoreMesh(core_axis_name="c", subcore_axis_name="s",
                                  num_cores=NC, num_subcores=NS)
    out_type = jax.ShapeDtypeStruct((E, W), jnp.float32)
    scratch = [pltpu.VMEM((_EB_SC,), jnp.int32),
               pltpu.VMEM((_EB_SC,), jnp.int32),
               pltpu.VMEM((_EB_SC, W), jnp.float32),
               pltpu.VMEM((_EB_SC, W), jnp.float32)] + \
              [pltpu.SemaphoreType.DMA] * 6

    @functools.partial(pl.kernel, out_type=out_type, mesh=mesh,
                       scratch_types=scratch)
    def k(h_hbm, idx2d, out_hbm, ib0, ib1, rb0, rb1, gs0, gs1, ss0, ss1,
          is0, is1):
        wid = lax.axis_index("s") * NC + lax.axis_index("c")
        bb = wid * bn
        _gather_pipelined(h_hbm, out_hbm, idx2d, ib0, ib1, bb, bn, rb0, rb1,
                          gs0, gs1, ss0, ss1, is0, is1)

    return k


@functools.lru_cache(maxsize=None)
def _sc_scatter(E, N):
    bn = E // _EB_SC // NW
    assert bn % 2 == 0
    rt = (N // NS) & ~7
    rem = N - NS * rt
    mesh = plsc.VectorSubcoreMesh(core_axis_name="c", subcore_axis_name="s",
                                  num_cores=NC, num_subcores=NS)
    out_type = jax.ShapeDtypeStruct((NC, N, 128), jnp.float32)
    scratch = [pltpu.VMEM((bn, _EB_SC), jnp.int32),
               pltpu.VMEM((_EB_SC, 128), jnp.float32),
               pltpu.VMEM((_EB_SC, 128), jnp.float32),
               pltpu.VMEM_SHARED((N, 128), jnp.float32)] + \
              [pltpu.SemaphoreType.DMA] * 4

    @functools.partial(pl.kernel, out_type=out_type, mesh=mesh,
                       scratch_types=scratch)
    def k(rows_hbm, dst2d, zeros_hbm, out_hbm, ibuf, rb0, rb1, acc, ls0,
          ls1, cs0, cs1):
        sid = lax.axis_index("s")
        cid = lax.axis_index("c")
        wid = sid * NC + cid
        bb = wid * bn
        r0 = sid * rt
        pltpu.sync_copy(zeros_hbm.at[pl.ds(r0, rt)], acc.at[pl.ds(r0, rt)])
        if rem:
            @pl.when(sid == NS - 1)
            def _():
                pltpu.sync_copy(zeros_hbm.at[pl.ds(NS * rt, rem)],
                                acc.at[pl.ds(NS * rt, rem)])
        pltpu.sync_copy(dst2d.at[pl.ds(bb, bn)], ibuf)
        plsc.subcore_barrier()

        def l_start(j, rb, sem):
            pltpu.async_copy(rows_hbm.at[pl.ds((bb + j) * _EB_SC, _EB_SC)],
                             rb, sem)

        def l_wait(rb, sem):
            pltpu.make_async_copy(rows_hbm.at[pl.ds(0, _EB_SC)], rb,
                                  sem).wait()

        def c_start(j, rb, sem):
            pltpu.async_copy(rb, acc.at[ibuf.at[j]], sem, add=True)

        def c_wait(rb, sem):
            pltpu.make_async_copy(rb, acc.at[ibuf.at[0]], sem).wait()

        l_start(0, rb0, ls0)

        def pair(p, _):
            j0 = 2 * p
            j1 = j0 + 1
            l_start(j1, rb1, ls1)
            l_wait(rb0, ls0)
            c_start(j0, rb0, cs0)
            l_wait(rb1, ls1)
            c_wait(rb0, cs0)
            c_start(j1, rb1, cs1)

            @pl.when(j0 + 2 < bn)
            def _():
                l_start(j0 + 2, rb0, ls0)

            c_wait(rb1, cs1)
            return 0

        lax.fori_loop(0, bn // 2, pair, 0)
        plsc.subcore_barrier()
        pltpu.sync_copy(acc.at[pl.ds(r0, rt)],
                        out_hbm.at[cid, pl.ds(r0, rt)])
        if rem:
            @pl.when(sid == NS - 1)
            def _():
                pltpu.sync_copy(acc.at[pl.ds(NS * rt, rem)],
                                out_hbm.at[cid, pl.ds(NS * rt, rem)])

    return k


_RB = 2000
_EB = 512


def _tc_embed(x, W, b):
    n, _ = x.shape
    h = W.shape[1]

    def body(x_ref, w_ref, b_ref, o_ref, o16_ref):
        v = _leaky(
            jnp.dot(x_ref[...], w_ref[...],
                    preferred_element_type=jnp.float32) + b_ref[...])
        o_ref[...] = v
        o16_ref[...] = v.astype(jnp.bfloat16)

    return pl.pallas_call(
        body,
        grid=(n // _RB,),
        in_specs=[
            pl.BlockSpec((_RB, x.shape[1]), lambda i: (i, 0)),
            pl.BlockSpec(W.shape, lambda i: (0, 0)),
            pl.BlockSpec((1, h), lambda i: (0, 0)),
        ],
        out_specs=[pl.BlockSpec((_RB, h), lambda i: (i, 0)),
                   pl.BlockSpec((_RB, h), lambda i: (i, 0))],
        out_shape=[jax.ShapeDtypeStruct((n, h), jnp.float32),
                   jax.ShapeDtypeStruct((n, h), jnp.bfloat16)],
    )(x, W, b.reshape(1, h))


def _tc_edge_mlp(hs_parts, hd_parts, w_in, W1_parts, b1, W2, b2):
    E = w_in.shape[0]
    P = len(hs_parts)
    hid = b1.shape[0]
    nchunks = sum(p.shape[1] // 128 for p in hs_parts)

    def body(*refs):
        hs = refs[0:P]
        hd = refs[P:2 * P]
        w_ref = refs[2 * P]
        W1 = refs[2 * P + 1:3 * P + 1]
        b1_ref, W2_ref, b2_ref = refs[3 * P + 1:3 * P + 4]
        w_out_ref = refs[3 * P + 4]
        r_refs = refs[3 * P + 5:]
        m = b1_ref[...]
        for p in range(P):
            d = jnp.abs(hs[p][...] - hd[p][...])
            m = m + jnp.dot(d, W1[p][...], preferred_element_type=jnp.float32)
        m = jnp.maximum(m, 0.0)
        s = jax.nn.sigmoid(
            jnp.dot(m, W2_ref[...], preferred_element_type=jnp.float32)
            + b2_ref[...])
        w = w_ref[...] * s[:, 0]
        w_out_ref[...] = w
        ci = 0
        for p in range(P):
            scaled = w[:, None] * hs[p][...].astype(jnp.float32)
            for c in range(hs[p].shape[1] // 128):
                r_refs[ci][...] = scaled[:, c * 128:(c + 1) * 128]
                ci += 1

    in_specs = (
        [pl.BlockSpec((_EB, p.shape[1]), lambda i: (i, 0)) for p in hs_parts]
        + [pl.BlockSpec((_EB, p.shape[1]), lambda i: (i, 0)) for p in hd_parts]
        + [pl.BlockSpec((_EB,), lambda i: (i,))]
        + [pl.BlockSpec(Wp.shape, lambda i: (0, 0)) for Wp in W1_parts]
        + [pl.BlockSpec((1, hid), lambda i: (0, 0)),
           pl.BlockSpec(W2.shape, lambda i: (0, 0)),
           pl.BlockSpec((1, 1), lambda i: (0, 0))]
    )
    out_specs = (
        [pl.BlockSpec((_EB,), lambda i: (i,))]
        + [pl.BlockSpec((_EB, 128), lambda i: (i, 0))] * nchunks
    )
    out_shape = (
        [jax.ShapeDtypeStruct((E,), jnp.float32)]
        + [jax.ShapeDtypeStruct((E, 128), jnp.float32)] * nchunks
    )
    res = pl.pallas_call(
        body,
        grid=(E // _EB,),
        in_specs=in_specs,
        out_specs=out_specs,
        out_shape=out_shape,
    )(*hs_parts, *hd_parts, w_in, *W1_parts, b1.reshape(1, hid), W2,
      b2.reshape(1, 1))
    return res[0], list(res[1:])


def _tc_dense_bn(h_parts, Wt_parts, agg_chunks, Wb_chunks, b):
    n = h_parts[0].shape[0]
    hid = b.shape[0]
    P = len(h_parts)
    C = len(agg_chunks)
    nblk = n // _RB

    def body(*refs):
        hp = refs[0:P]
        Wt = refs[P:2 * P]
        ag = refs[2 * P:2 * P + C]
        Wb = refs[2 * P + C:2 * P + 2 * C]
        b_ref = refs[2 * P + 2 * C]
        y_ref, mu_ref, var_ref = refs[2 * P + 2 * C + 1:2 * P + 2 * C + 4]
        s_ref, q_ref = refs[2 * P + 2 * C + 4:]
        i = pl.program_id(0)
        y = b_ref[...]
        for p in range(P):
            y = y + jnp.dot(hp[p][...], Wt[p][...],
                            preferred_element_type=jnp.float32)
        for c in range(C):
            a = ag[c][0] + ag[c][1]
            y = y + jnp.dot(a, Wb[c][...], preferred_element_type=jnp.float32)
        y_ref[...] = y

        @pl.when(i == 0)
        def _():
            s_ref[...] = jnp.zeros_like(s_ref)
            q_ref[...] = jnp.zeros_like(q_ref)

        s_ref[...] += jnp.sum(y, axis=0, keepdims=True)
        q_ref[...] += jnp.sum(y * y, axis=0, keepdims=True)

        @pl.when(i == nblk - 1)
        def _():
            mu = s_ref[...] / n
            mu_ref[...] = mu
            var_ref[...] = q_ref[...] / n - mu * mu

    in_specs = (
        [pl.BlockSpec((_RB, p.shape[1]), lambda i: (i, 0)) for p in h_parts]
        + [pl.BlockSpec(Wp.shape, lambda i: (0, 0)) for Wp in Wt_parts]
        + [pl.BlockSpec((NC, _RB, 128), lambda i: (0, i, 0))
           for _ in agg_chunks]
        + [pl.BlockSpec(Wc.shape, lambda i: (0, 0)) for Wc in Wb_chunks]
        + [pl.BlockSpec((1, hid), lambda i: (0, 0))]
    )
    out_specs = [pl.BlockSpec((_RB, hid), lambda i: (i, 0)),
                 pl.BlockSpec((1, hid), lambda i: (0, 0)),
                 pl.BlockSpec((1, hid), lambda i: (0, 0))]
    out_shape = [jax.ShapeDtypeStruct((n, hid), jnp.float32),
                 jax.ShapeDtypeStruct((1, hid), jnp.float32),
                 jax.ShapeDtypeStruct((1, hid), jnp.float32)]
    y, mu, var = pl.pallas_call(
        body,
        grid=(nblk,),
        in_specs=in_specs,
        out_specs=out_specs,
        out_shape=out_shape,
        scratch_shapes=[pltpu.VMEM((1, hid), jnp.float32),
                        pltpu.VMEM((1, hid), jnp.float32)],
    )(*h_parts, *Wt_parts, *agg_chunks, *Wb_chunks, b.reshape(1, hid))
    return y, mu, var


def _tc_bn_apply(y, mu, var, gamma, beta):
    n, hid = y.shape

    def body(y_ref, mu_ref, var_ref, g_ref, be_ref, o_ref, o16_ref):
        inv = g_ref[...] * lax.rsqrt(var_ref[...] + 1e-5)
        v = _leaky((y_ref[...] - mu_ref[...]) * inv + be_ref[...])
        o_ref[...] = v
        o16_ref[...] = v.astype(jnp.bfloat16)

    return pl.pallas_call(
        body,
        grid=(n // _RB,),
        in_specs=[pl.BlockSpec((_RB, hid), lambda i: (i, 0)),
                  pl.BlockSpec((1, hid), lambda i: (0, 0)),
                  pl.BlockSpec((1, hid), lambda i: (0, 0)),
                  pl.BlockSpec((1, hid), lambda i: (0, 0)),
                  pl.BlockSpec((1, hid), lambda i: (0, 0))],
        out_specs=[pl.BlockSpec((_RB, hid), lambda i: (i, 0)),
                   pl.BlockSpec((_RB, hid), lambda i: (i, 0))],
        out_shape=[jax.ShapeDtypeStruct((n, hid), jnp.float32),
                   jax.ShapeDtypeStruct((n, hid), jnp.bfloat16)],
    )(y, mu, var, gamma.reshape(1, hid), beta.reshape(1, hid))


def _tc_matmul_parts(h_parts, W_parts, out_w, extra=None,
                     out_dtype=jnp.float32):
    n = h_parts[0].shape[0]
    P = len(h_parts)
    has_extra = extra is not None

    def body(*refs):
        hp = refs[0:P]
        Wp = refs[P:2 * P]
        k = 2 * P
        acc = jnp.zeros((_RB, out_w), jnp.float32)
        if has_extra:
            acc = refs[k][0] + refs[k][1]
            k += 1
        o_ref = refs[k]
        for p in range(P):
            acc = acc + jnp.dot(hp[p][...], Wp[p][...],
                                preferred_element_type=jnp.float32)
        o_ref[...] = acc.astype(out_dtype)

    in_specs = (
        [pl.BlockSpec((_RB, p.shape[1]), lambda i: (i, 0)) for p in h_parts]
        + [pl.BlockSpec(Wp.shape, lambda i: (0, 0)) for Wp in W_parts]
    )
    args = list(h_parts) + list(W_parts)
    if has_extra:
        in_specs.append(pl.BlockSpec((NC, _RB, out_w), lambda i: (0, i, 0)))
        args.append(extra)
    return pl.pallas_call(
        body,
        grid=(n // _RB,),
        in_specs=in_specs,
        out_specs=pl.BlockSpec((_RB, out_w), lambda i: (i, 0)),
        out_shape=jax.ShapeDtypeStruct((n, out_w), out_dtype),
    )(*args)


def _tc_scale_rows(rows, w):
    E, width = rows.shape

    def body(r_ref, w_ref, o_ref):
        o_ref[...] = w_ref[...][:, None] * r_ref[...].astype(jnp.float32)

    return pl.pallas_call(
        body,
        grid=(E // _EB,),
        in_specs=[pl.BlockSpec((_EB, width), lambda i: (i, 0)),
                  pl.BlockSpec((_EB,), lambda i: (i,))],
        out_specs=pl.BlockSpec((_EB, width), lambda i: (i, 0)),
        out_shape=jax.ShapeDtypeStruct((E, width), jnp.float32),
    )(rows, w)


def kernel(x, edge_index, edge_weight, g_size, emb_W, emb_b, wc1_W1, wc1_b1,
           wc1_W2, wc1_b2, gc1_W, gc1_b, gc1_g, gc1_be, wc2_W1, wc2_b1,
           wc2_W2, wc2_b2, gc2_W, gc2_b, gc2_g, gc2_be, gcl_W):
    N, _ = x.shape
    E = edge_index.shape[1]
    HID = emb_W.shape[1]
    chunk = NW * _EB_SC * 2
    EP = ((E + chunk - 1) // chunk) * chunk
    pad_idx = (jnp.arange(EP - E, dtype=jnp.int32) * 8) % N
    src = jnp.concatenate([edge_index[0], pad_idx]).reshape(
        EP // _EB_SC, _EB_SC)
    dst = jnp.concatenate([edge_index[1], pad_idx]).reshape(
        EP // _EB_SC, _EB_SC)
    ew = jnp.pad(edge_weight, (0, EP - E))
    zeros128 = jnp.zeros((N, 128), jnp.float32)

    h0, h0b = _tc_embed(x, emb_W, emb_b)

    g2p = _sc_gather2(EP, N, HID // 2)
    hs0p, hd0p = g2p(_pack_bf16(h0b), src, dst)
    hs0 = _unpack_bf16(hs0p)
    hd0 = _unpack_bf16(hd0p)
    w1, r1 = _tc_edge_mlp([hs0], [hd0], ew,
                          [wc1_W1.astype(jnp.bfloat16)], wc1_b1,
                          wc1_W2, wc1_b2)
    aggs1 = [_sc_scatter(EP, N)(r, dst, zeros128) for r in r1]
    y1, mu1, var1 = _tc_dense_bn(
        [h0], [gc1_W[:HID]], aggs1,
        [gc1_W[HID + 128 * c:HID + 128 * (c + 1)] for c in range(2)], gc1_b)
    hn1, hn1b = _tc_bn_apply(y1, mu1, var1, gc1_g, gc1_be)

    hsnp, hdnp = g2p(_pack_bf16(hn1b), src, dst)
    hsn = _unpack_bf16(hsnp)
    hdn = _unpack_bf16(hdnp)
    w2, r2 = _tc_edge_mlp([hs0, hsn], [hd0, hdn], w1,
                          [wc2_W1[:HID].astype(jnp.bfloat16),
                           wc2_W1[HID:].astype(jnp.bfloat16)],
                          wc2_b1, wc2_W2, wc2_b2)
    aggs2 = [_sc_scatter(EP, N)(r, dst, zeros128) for r in r2]
    y2, mu2, var2 = _tc_dense_bn(
        [h0, hn1], [gc2_W[:HID], gc2_W[HID:2 * HID]], aggs2,
        [gc2_W[2 * HID + 128 * c:2 * HID + 128 * (c + 1)] for c in range(4)],
        gc2_b)
    hn2, _hn2b = _tc_bn_apply(y2, mu2, var2, gc2_g, gc2_be)

    OUT = gcl_W.shape[1]
    h2_parts = [h0, hn1, hn2]
    Wb = [gcl_W[3 * HID + HID * p:3 * HID + HID * (p + 1)] for p in range(3)]
    z = _tc_matmul_parts(h2_parts, Wb, OUT)
    zs = _sc_gather1(EP, N, OUT)(z, src)
    rfin = _tc_scale_rows(zs, w2)
    aggf = _sc_scatter(EP, N)(rfin, dst, zeros128)
    Wt = [gcl_W[HID * p:HID * (p + 1)] for p in range(3)]
    out = _tc_matmul_parts(h2_parts, Wt, OUT, extra=aggf)

    return (out, w2[:E], g_size)


# speedup vs baseline: 2.4657x; 2.4657x over previous
import functools

import jax
import jax.numpy as jnp
from jax import lax
from jax.experimental import pallas as pl
from jax.experimental.pallas import tpu as pltpu
from jax.experimental.pallas import tpu_sc as plsc

NC = 2
NS = 16
NW = NC * NS

_EB_SC = 128


def _leaky(v):
    return jnp.where(v >= 0, v, 0.01 * v)


def _gather_pipelined(h_hbm, out_hbm, idx2d, ib0, ib1, bb, bn, rb0, rb1,
                      gs0, gs1, ss0, ss1, is0, is1):
    def i_start(j, ib, sem):
        pltpu.async_copy(idx2d.at[bb + j], ib, sem)

    def i_wait(ib, sem):
        pltpu.make_async_copy(idx2d.at[0], ib, sem).wait()

    def g_start(ib, rb, sem):
        pltpu.async_copy(h_hbm.at[ib], rb, sem)

    def g_wait(ib, rb, sem):
        pltpu.make_async_copy(h_hbm.at[ib], rb, sem).wait()

    def s_start(j, rb, sem):
        pltpu.async_copy(rb, out_hbm.at[pl.ds((bb + j) * _EB_SC, _EB_SC)],
                         sem)

    def s_wait(rb, sem):
        pltpu.make_async_copy(rb, out_hbm.at[pl.ds(0, _EB_SC)], sem).wait()

    i_start(0, ib0, is0)

    def pair(p, _):
        j0 = 2 * p
        j1 = j0 + 1
        i_start(j1, ib1, is1)
        i_wait(ib0, is0)
        g_start(ib0, rb0, gs0)
        g_wait(ib0, rb0, gs0)
        s_start(j0, rb0, ss0)
        i_wait(ib1, is1)
        g_start(ib1, rb1, gs1)

        @pl.when(j0 + 2 < bn)
        def _():
            i_start(j0 + 2, ib0, is0)

        g_wait(ib1, rb1, gs1)
        s_wait(rb0, ss0)
        s_start(j1, rb1, ss1)
        s_wait(rb1, ss1)
        return 0

    lax.fori_loop(0, bn // 2, pair, 0)


def _pack_in_kernel(v):
    w = v.shape[1]
    lo = v[:, :w // 2].astype(jnp.bfloat16).astype(jnp.float32)
    hi = v[:, w // 2:].astype(jnp.bfloat16).astype(jnp.float32)
    lo_b = lax.bitcast_convert_type(lo, jnp.uint32) >> 16
    hi_b = lax.bitcast_convert_type(hi, jnp.uint32) & jnp.uint32(0xFFFF0000)
    return lax.bitcast_convert_type(lo_b | hi_b, jnp.float32)


def _unpack_halves(p):
    u = lax.bitcast_convert_type(p, jnp.uint32)
    lo = lax.bitcast_convert_type(u << 16, jnp.float32)
    hi = lax.bitcast_convert_type(u & jnp.uint32(0xFFFF0000), jnp.float32)
    return lo, hi


@functools.lru_cache(maxsize=None)
def _sc_gather2(E, N, W):
    bn = E // _EB_SC // NW
    assert bn % 2 == 0
    mesh = plsc.VectorSubcoreMesh(core_axis_name="c", subcore_axis_name="s",
                                  num_cores=NC, num_subcores=NS)
    out_type = [jax.ShapeDtypeStruct((E, W), jnp.float32),
                jax.ShapeDtypeStruct((E, W), jnp.float32)]
    scratch = [pltpu.VMEM((_EB_SC,), jnp.int32),
               pltpu.VMEM((_EB_SC,), jnp.int32),
               pltpu.VMEM((_EB_SC, W), jnp.float32),
               pltpu.VMEM((_EB_SC, W), jnp.float32)] + \
              [pltpu.SemaphoreType.DMA] * 6

    @functools.partial(pl.kernel, out_type=out_type, mesh=mesh,
                       scratch_types=scratch)
    def k(h_hbm, src2d, dst2d, hs_hbm, hd_hbm, ib0, ib1, rb0, rb1, gs0, gs1,
          ss0, ss1, is0, is1):
        wid = lax.axis_index("s") * NC + lax.axis_index("c")
        bb = wid * bn
        for idx2d, out_hbm in ((src2d, hs_hbm), (dst2d, hd_hbm)):
            _gather_pipelined(h_hbm, out_hbm, idx2d, ib0, ib1, bb, bn, rb0,
                              rb1, gs0, gs1, ss0, ss1, is0, is1)

    return k


@functools.lru_cache(maxsize=None)
def _sc_gather1(E, N, W):
    bn = E // _EB_SC // NW
    assert bn % 2 == 0
    mesh = plsc.VectorSubcoreMesh(core_axis_name="c", subcore_axis_name="s",
                                  num_cores=NC, num_subcores=NS)
    out_type = jax.ShapeDtypeStruct((E, W), jnp.float32)
    scratch = [pltpu.VMEM((_EB_SC,), jnp.int32),
               pltpu.VMEM((_EB_SC,), jnp.int32),
               pltpu.VMEM((_EB_SC, W), jnp.float32),
               pltpu.VMEM((_EB_SC, W), jnp.float32)] + \
              [pltpu.SemaphoreType.DMA] * 6

    @functools.partial(pl.kernel, out_type=out_type, mesh=mesh,
                       scratch_types=scratch)
    def k(h_hbm, idx2d, out_hbm, ib0, ib1, rb0, rb1, gs0, gs1, ss0, ss1,
          is0, is1):
        wid = lax.axis_index("s") * NC + lax.axis_index("c")
        bb = wid * bn
        _gather_pipelined(h_hbm, out_hbm, idx2d, ib0, ib1, bb, bn, rb0, rb1,
                          gs0, gs1, ss0, ss1, is0, is1)

    return k


@functools.lru_cache(maxsize=None)
def _sc_scatter(E, N):
    bn = E // _EB_SC // NW
    assert bn % 2 == 0
    rt = (N // NS) & ~7
    rem = N - NS * rt
    mesh = plsc.VectorSubcoreMesh(core_axis_name="c", subcore_axis_name="s",
                                  num_cores=NC, num_subcores=NS)
    out_type = jax.ShapeDtypeStruct((NC, N, 128), jnp.float32)
    scratch = [pltpu.VMEM((bn, _EB_SC), jnp.int32),
               pltpu.VMEM((_EB_SC, 128), jnp.float32),
               pltpu.VMEM((_EB_SC, 128), jnp.float32),
               pltpu.VMEM_SHARED((N, 128), jnp.float32)] + \
              [pltpu.SemaphoreType.DMA] * 4

    @functools.partial(pl.kernel, out_type=out_type, mesh=mesh,
                       scratch_types=scratch)
    def k(rows_hbm, dst2d, zeros_hbm, out_hbm, ibuf, rb0, rb1, acc, ls0,
          ls1, cs0, cs1):
        sid = lax.axis_index("s")
        cid = lax.axis_index("c")
        wid = sid * NC + cid
        bb = wid * bn
        r0 = sid * rt
        pltpu.sync_copy(zeros_hbm.at[pl.ds(r0, rt)], acc.at[pl.ds(r0, rt)])
        if rem:
            @pl.when(sid == NS - 1)
            def _():
                pltpu.sync_copy(zeros_hbm.at[pl.ds(NS * rt, rem)],
                                acc.at[pl.ds(NS * rt, rem)])
        pltpu.sync_copy(dst2d.at[pl.ds(bb, bn)], ibuf)
        plsc.subcore_barrier()

        def l_start(j, rb, sem):
            pltpu.async_copy(rows_hbm.at[pl.ds((bb + j) * _EB_SC, _EB_SC)],
                             rb, sem)

        def l_wait(rb, sem):
            pltpu.make_async_copy(rows_hbm.at[pl.ds(0, _EB_SC)], rb,
                                  sem).wait()

        def c_start(j, rb, sem):
            pltpu.async_copy(rb, acc.at[ibuf.at[j]], sem, add=True)

        def c_wait(rb, sem):
            pltpu.make_async_copy(rb, acc.at[ibuf.at[0]], sem).wait()

        l_start(0, rb0, ls0)

        def pair(p, _):
            j0 = 2 * p
            j1 = j0 + 1
            l_start(j1, rb1, ls1)
            l_wait(rb0, ls0)
            c_start(j0, rb0, cs0)
            l_wait(rb1, ls1)
            c_wait(rb0, cs0)
            c_start(j1, rb1, cs1)

            @pl.when(j0 + 2 < bn)
            def _():
                l_start(j0 + 2, rb0, ls0)

            c_wait(rb1, cs1)
            return 0

        lax.fori_loop(0, bn // 2, pair, 0)
        plsc.subcore_barrier()
        pltpu.sync_copy(acc.at[pl.ds(r0, rt)],
                        out_hbm.at[cid, pl.ds(r0, rt)])
        if rem:
            @pl.when(sid == NS - 1)
            def _():
                pltpu.sync_copy(acc.at[pl.ds(NS * rt, rem)],
                                out_hbm.at[cid, pl.ds(NS * rt, rem)])

    return k


_RB = 2000
_EB = 512


def _tc_embed(x, W, b):
    n, _ = x.shape
    h = W.shape[1]

    def body(x_ref, w_ref, b_ref, o_ref, o16_ref):
        v = _leaky(
            jnp.dot(x_ref[...], w_ref[...],
                    preferred_element_type=jnp.float32) + b_ref[...])
        o_ref[...] = v
        o16_ref[...] = _pack_in_kernel(v)

    return pl.pallas_call(
        body,
        grid=(n // _RB,),
        in_specs=[
            pl.BlockSpec((_RB, x.shape[1]), lambda i: (i, 0)),
            pl.BlockSpec(W.shape, lambda i: (0, 0)),
            pl.BlockSpec((1, h), lambda i: (0, 0)),
        ],
        out_specs=[pl.BlockSpec((_RB, h), lambda i: (i, 0)),
                   pl.BlockSpec((_RB, h // 2), lambda i: (i, 0))],
        out_shape=[jax.ShapeDtypeStruct((n, h), jnp.float32),
                   jax.ShapeDtypeStruct((n, h // 2), jnp.float32)],
    )(x, W, b.reshape(1, h))


def _tc_edge_mlp(hs_parts, hd_parts, w_in, W1_parts, b1, W2, b2):
    E = w_in.shape[0]
    P = len(hs_parts)
    hid = b1.shape[0]
    nchunks = sum(2 * p.shape[1] // 128 for p in hs_parts)

    def body(*refs):
        hs = refs[0:P]
        hd = refs[P:2 * P]
        w_ref = refs[2 * P]
        W1 = refs[2 * P + 1:3 * P + 1]
        b1_ref, W2_ref, b2_ref = refs[3 * P + 1:3 * P + 4]
        w_out_ref = refs[3 * P + 4]
        r_refs = refs[3 * P + 5:]
        m = b1_ref[...]
        hs_un = []
        for p in range(P):
            alo, ahi = _unpack_halves(hs[p][...])
            blo, bhi = _unpack_halves(hd[p][...])
            hs_un.append((alo, ahi))
            d = jnp.concatenate(
                [jnp.abs(alo - blo), jnp.abs(ahi - bhi)], axis=1)
            m = m + jnp.dot(d, W1[p][...], preferred_element_type=jnp.float32)
        m = jnp.maximum(m, 0.0)
        s = jax.nn.sigmoid(
            jnp.dot(m, W2_ref[...], preferred_element_type=jnp.float32)
            + b2_ref[...])
        w = w_ref[...] * s[:, 0]
        w_out_ref[...] = w
        ci = 0
        for p in range(P):
            for half in hs_un[p]:
                r_refs[ci][...] = w[:, None] * half
                ci += 1

    in_specs = (
        [pl.BlockSpec((_EB, p.shape[1]), lambda i: (i, 0)) for p in hs_parts]
        + [pl.BlockSpec((_EB, p.shape[1]), lambda i: (i, 0)) for p in hd_parts]
        + [pl.BlockSpec((_EB,), lambda i: (i,))]
        + [pl.BlockSpec(Wp.shape, lambda i: (0, 0)) for Wp in W1_parts]
        + [pl.BlockSpec((1, hid), lambda i: (0, 0)),
           pl.BlockSpec(W2.shape, lambda i: (0, 0)),
           pl.BlockSpec((1, 1), lambda i: (0, 0))]
    )
    out_specs = (
        [pl.BlockSpec((_EB,), lambda i: (i,))]
        + [pl.BlockSpec((_EB, 128), lambda i: (i, 0))] * nchunks
    )
    out_shape = (
        [jax.ShapeDtypeStruct((E,), jnp.float32)]
        + [jax.ShapeDtypeStruct((E, 128), jnp.float32)] * nchunks
    )
    res = pl.pallas_call(
        body,
        grid=(E // _EB,),
        in_specs=in_specs,
        out_specs=out_specs,
        out_shape=out_shape,
    )(*hs_parts, *hd_parts, w_in, *W1_parts, b1.reshape(1, hid), W2,
      b2.reshape(1, 1))
    return res[0], list(res[1:])


def _tc_dense_bn(h_parts, Wt_parts, agg_chunks, Wb_chunks, b):
    n = h_parts[0].shape[0]
    hid = b.shape[0]
    P = len(h_parts)
    C = len(agg_chunks)
    nblk = n // _RB

    def body(*refs):
        hp = refs[0:P]
        Wt = refs[P:2 * P]
        ag = refs[2 * P:2 * P + C]
        Wb = refs[2 * P + C:2 * P + 2 * C]
        b_ref = refs[2 * P + 2 * C]
        y_ref, mu_ref, var_ref = refs[2 * P + 2 * C + 1:2 * P + 2 * C + 4]
        s_ref, q_ref = refs[2 * P + 2 * C + 4:]
        i = pl.program_id(0)
        y = b_ref[...]
        for p in range(P):
            y = y + jnp.dot(hp[p][...], Wt[p][...],
                            preferred_element_type=jnp.float32)
        for c in range(C):
            a = ag[c][0] + ag[c][1]
            y = y + jnp.dot(a, Wb[c][...], preferred_element_type=jnp.float32)
        y_ref[...] = y

        @pl.when(i == 0)
        def _():
            s_ref[...] = jnp.zeros_like(s_ref)
            q_ref[...] = jnp.zeros_like(q_ref)

        s_ref[...] += jnp.sum(y, axis=0, keepdims=True)
        q_ref[...] += jnp.sum(y * y, axis=0, keepdims=True)

        @pl.when(i == nblk - 1)
        def _():
            mu = s_ref[...] / n
            mu_ref[...] = mu
            var_ref[...] = q_ref[...] / n - mu * mu

    in_specs = (
        [pl.BlockSpec((_RB, p.shape[1]), lambda i: (i, 0)) for p in h_parts]
        + [pl.BlockSpec(Wp.shape, lambda i: (0, 0)) for Wp in Wt_parts]
        + [pl.BlockSpec((NC, _RB, 128), lambda i: (0, i, 0))
           for _ in agg_chunks]
        + [pl.BlockSpec(Wc.shape, lambda i: (0, 0)) for Wc in Wb_chunks]
        + [pl.BlockSpec((1, hid), lambda i: (0, 0))]
    )
    out_specs = [pl.BlockSpec((_RB, hid), lambda i: (i, 0)),
                 pl.BlockSpec((1, hid), lambda i: (0, 0)),
                 pl.BlockSpec((1, hid), lambda i: (0, 0))]
    out_shape = [jax.ShapeDtypeStruct((n, hid), jnp.float32),
                 jax.ShapeDtypeStruct((1, hid), jnp.float32),
                 jax.ShapeDtypeStruct((1, hid), jnp.float32)]
    y, mu, var = pl.pallas_call(
        body,
        grid=(nblk,),
        in_specs=in_specs,
        out_specs=out_specs,
        out_shape=out_shape,
        scratch_shapes=[pltpu.VMEM((1, hid), jnp.float32),
                        pltpu.VMEM((1, hid), jnp.float32)],
    )(*h_parts, *Wt_parts, *agg_chunks, *Wb_chunks, b.reshape(1, hid))
    return y, mu, var


def _tc_bn_apply(y, mu, var, gamma, beta):
    n, hid = y.shape

    def body(y_ref, mu_ref, var_ref, g_ref, be_ref, o_ref, o16_ref):
        inv = g_ref[...] * lax.rsqrt(var_ref[...] + 1e-5)
        v = _leaky((y_ref[...] - mu_ref[...]) * inv + be_ref[...])
        o_ref[...] = v
        o16_ref[...] = _pack_in_kernel(v)

    return pl.pallas_call(
        body,
        grid=(n // _RB,),
        in_specs=[pl.BlockSpec((_RB, hid), lambda i: (i, 0)),
                  pl.BlockSpec((1, hid), lambda i: (0, 0)),
                  pl.BlockSpec((1, hid), lambda i: (0, 0)),
                  pl.BlockSpec((1, hid), lambda i: (0, 0)),
                  pl.BlockSpec((1, hid), lambda i: (0, 0))],
        out_specs=[pl.BlockSpec((_RB, hid), lambda i: (i, 0)),
                   pl.BlockSpec((_RB, hid // 2), lambda i: (i, 0))],
        out_shape=[jax.ShapeDtypeStruct((n, hid), jnp.float32),
                   jax.ShapeDtypeStruct((n, hid // 2), jnp.float32)],
    )(y, mu, var, gamma.reshape(1, hid), beta.reshape(1, hid))


def _tc_matmul_parts(h_parts, W_parts, out_w, extra=None,
                     out_dtype=jnp.float32):
    n = h_parts[0].shape[0]
    P = len(h_parts)
    has_extra = extra is not None

    def body(*refs):
        hp = refs[0:P]
        Wp = refs[P:2 * P]
        k = 2 * P
        acc = jnp.zeros((_RB, out_w), jnp.float32)
        if has_extra:
            acc = refs[k][0] + refs[k][1]
            k += 1
        o_ref = refs[k]
        for p in range(P):
            acc = acc + jnp.dot(hp[p][...], Wp[p][...],
                                preferred_element_type=jnp.float32)
        o_ref[...] = acc.astype(out_dtype)

    in_specs = (
        [pl.BlockSpec((_RB, p.shape[1]), lambda i: (i, 0)) for p in h_parts]
        + [pl.BlockSpec(Wp.shape, lambda i: (0, 0)) for Wp in W_parts]
    )
    args = list(h_parts) + list(W_parts)
    if has_extra:
        in_specs.append(pl.BlockSpec((NC, _RB, out_w), lambda i: (0, i, 0)))
        args.append(extra)
    return pl.pallas_call(
        body,
        grid=(n // _RB,),
        in_specs=in_specs,
        out_specs=pl.BlockSpec((_RB, out_w), lambda i: (i, 0)),
        out_shape=jax.ShapeDtypeStruct((n, out_w), out_dtype),
    )(*args)


def _tc_scale_rows(rows, w):
    E, width = rows.shape

    def body(r_ref, w_ref, o_ref):
        o_ref[...] = w_ref[...][:, None] * r_ref[...].astype(jnp.float32)

    return pl.pallas_call(
        body,
        grid=(E // _EB,),
        in_specs=[pl.BlockSpec((_EB, width), lambda i: (i, 0)),
                  pl.BlockSpec((_EB,), lambda i: (i,))],
        out_specs=pl.BlockSpec((_EB, width), lambda i: (i, 0)),
        out_shape=jax.ShapeDtypeStruct((E, width), jnp.float32),
    )(rows, w)


def kernel(x, edge_index, edge_weight, g_size, emb_W, emb_b, wc1_W1, wc1_b1,
           wc1_W2, wc1_b2, gc1_W, gc1_b, gc1_g, gc1_be, wc2_W1, wc2_b1,
           wc2_W2, wc2_b2, gc2_W, gc2_b, gc2_g, gc2_be, gcl_W):
    N, _ = x.shape
    E = edge_index.shape[1]
    HID = emb_W.shape[1]
    chunk = NW * _EB_SC * 2
    EP = ((E + chunk - 1) // chunk) * chunk
    pad_idx = (jnp.arange(EP - E, dtype=jnp.int32) * 8) % N
    src = jnp.concatenate([edge_index[0], pad_idx]).reshape(
        EP // _EB_SC, _EB_SC)
    dst = jnp.concatenate([edge_index[1], pad_idx]).reshape(
        EP // _EB_SC, _EB_SC)
    ew = jnp.pad(edge_weight, (0, EP - E))
    zeros128 = jnp.zeros((N, 128), jnp.float32)

    h0, h0b = _tc_embed(x, emb_W, emb_b)

    g2p = _sc_gather2(EP, N, HID // 2)
    hs0, hd0 = g2p(h0b, src, dst)
    w1, r1 = _tc_edge_mlp([hs0], [hd0], ew, [wc1_W1], wc1_b1,
                          wc1_W2, wc1_b2)
    aggs1 = [_sc_scatter(EP, N)(r, dst, zeros128) for r in r1]
    y1, mu1, var1 = _tc_dense_bn(
        [h0], [gc1_W[:HID]], aggs1,
        [gc1_W[HID + 128 * c:HID + 128 * (c + 1)] for c in range(2)], gc1_b)
    hn1, hn1b = _tc_bn_apply(y1, mu1, var1, gc1_g, gc1_be)

    hsn, hdn = g2p(hn1b, src, dst)
    w2, r2 = _tc_edge_mlp([hs0, hsn], [hd0, hdn], w1,
                          [wc2_W1[:HID], wc2_W1[HID:]],
                          wc2_b1, wc2_W2, wc2_b2)
    aggs2 = [_sc_scatter(EP, N)(r, dst, zeros128) for r in r2]
    y2, mu2, var2 = _tc_dense_bn(
        [h0, hn1], [gc2_W[:HID], gc2_W[HID:2 * HID]], aggs2,
        [gc2_W[2 * HID + 128 * c:2 * HID + 128 * (c + 1)] for c in range(4)],
        gc2_b)
    hn2, _hn2b = _tc_bn_apply(y2, mu2, var2, gc2_g, gc2_be)

    OUT = gcl_W.shape[1]
    h2_parts = [h0, hn1, hn2]
    Wb = [gcl_W[3 * HID + HID * p:3 * HID + HID * (p + 1)] for p in range(3)]
    z = _tc_matmul_parts(h2_parts, Wb, OUT)
    zs = _sc_gather1(EP, N, OUT)(z, src)
    rfin = _tc_scale_rows(zs, w2)
    aggf = _sc_scatter(EP, N)(rfin, dst, zeros128)
    Wt = [gcl_W[HID * p:HID * (p + 1)] for p in range(3)]
    out = _tc_matmul_parts(h2_parts, Wt, OUT, extra=aggf)

    return (out, w2[:E], g_size)


# speedup vs baseline: 2.4699x; 1.0017x over previous
import functools

import jax
import jax.numpy as jnp
from jax import lax
from jax.experimental import pallas as pl
from jax.experimental.pallas import tpu as pltpu
from jax.experimental.pallas import tpu_sc as plsc

NC = 2
NS = 16
NW = NC * NS

_EB_SC = 128


def _leaky(v):
    return jnp.where(v >= 0, v, 0.01 * v)


def _gather_pipelined(h_hbm, out_hbm, idx2d, ib0, ib1, bb, bn, rb0, rb1,
                      gs0, gs1, ss0, ss1, is0, is1):
    def i_start(j, ib, sem):
        pltpu.async_copy(idx2d.at[bb + j], ib, sem)

    def i_wait(ib, sem):
        pltpu.make_async_copy(idx2d.at[0], ib, sem).wait()

    def g_start(ib, rb, sem):
        pltpu.async_copy(h_hbm.at[ib], rb, sem)

    def g_wait(ib, rb, sem):
        pltpu.make_async_copy(h_hbm.at[ib], rb, sem).wait()

    def s_start(j, rb, sem):
        pltpu.async_copy(rb, out_hbm.at[pl.ds((bb + j) * _EB_SC, _EB_SC)],
                         sem)

    def s_wait(rb, sem):
        pltpu.make_async_copy(rb, out_hbm.at[pl.ds(0, _EB_SC)], sem).wait()

    i_start(0, ib0, is0)

    def pair(p, _):
        j0 = 2 * p
        j1 = j0 + 1
        i_start(j1, ib1, is1)
        i_wait(ib0, is0)
        g_start(ib0, rb0, gs0)
        g_wait(ib0, rb0, gs0)
        s_start(j0, rb0, ss0)
        i_wait(ib1, is1)
        g_start(ib1, rb1, gs1)

        @pl.when(j0 + 2 < bn)
        def _():
            i_start(j0 + 2, ib0, is0)

        g_wait(ib1, rb1, gs1)
        s_wait(rb0, ss0)
        s_start(j1, rb1, ss1)
        s_wait(rb1, ss1)
        return 0

    lax.fori_loop(0, bn // 2, pair, 0)


def _pack_in_kernel(v):
    w = v.shape[1]
    lo = v[:, :w // 2].astype(jnp.bfloat16).astype(jnp.float32)
    hi = v[:, w // 2:].astype(jnp.bfloat16).astype(jnp.float32)
    lo_b = lax.bitcast_convert_type(lo, jnp.uint32) >> 16
    hi_b = lax.bitcast_convert_type(hi, jnp.uint32) & jnp.uint32(0xFFFF0000)
    return lax.bitcast_convert_type(lo_b | hi_b, jnp.float32)


def _unpack_halves(p):
    u = lax.bitcast_convert_type(p, jnp.uint32)
    lo = lax.bitcast_convert_type(u << 16, jnp.float32)
    hi = lax.bitcast_convert_type(u & jnp.uint32(0xFFFF0000), jnp.float32)
    return lo, hi


@functools.lru_cache(maxsize=None)
def _sc_gather2(E, N, W):
    bn = E // _EB_SC // NW
    assert bn % 2 == 0
    mesh = plsc.VectorSubcoreMesh(core_axis_name="c", subcore_axis_name="s",
                                  num_cores=NC, num_subcores=NS)
    out_type = [jax.ShapeDtypeStruct((E, W), jnp.float32),
                jax.ShapeDtypeStruct((E, W), jnp.float32)]
    scratch = [pltpu.VMEM((_EB_SC,), jnp.int32),
               pltpu.VMEM((_EB_SC,), jnp.int32),
               pltpu.VMEM((_EB_SC, W), jnp.float32),
               pltpu.VMEM((_EB_SC, W), jnp.float32)] + \
              [pltpu.SemaphoreType.DMA] * 6

    @functools.partial(pl.kernel, out_type=out_type, mesh=mesh,
                       scratch_types=scratch)
    def k(h_hbm, src2d, dst2d, hs_hbm, hd_hbm, ib0, ib1, rb0, rb1, gs0, gs1,
          ss0, ss1, is0, is1):
        wid = lax.axis_index("s") * NC + lax.axis_index("c")
        bb = wid * bn
        for idx2d, out_hbm in ((src2d, hs_hbm), (dst2d, hd_hbm)):
            _gather_pipelined(h_hbm, out_hbm, idx2d, ib0, ib1, bb, bn, rb0,
                              rb1, gs0, gs1, ss0, ss1, is0, is1)

    return k


@functools.lru_cache(maxsize=None)
def _sc_gather1(E, N, W):
    bn = E // _EB_SC // NW
    assert bn % 2 == 0
    mesh = plsc.VectorSubcoreMesh(core_axis_name="c", subcore_axis_name="s",
                                  num_cores=NC, num_subcores=NS)
    out_type = jax.ShapeDtypeStruct((E, W), jnp.float32)
    scratch = [pltpu.VMEM((_EB_SC,), jnp.int32),
               pltpu.VMEM((_EB_SC,), jnp.int32),
               pltpu.VMEM((_EB_SC, W), jnp.float32),
               pltpu.VMEM((_EB_SC, W), jnp.float32)] + \
              [pltpu.SemaphoreType.DMA] * 6

    @functools.partial(pl.kernel, out_type=out_type, mesh=mesh,
                       scratch_types=scratch)
    def k(h_hbm, idx2d, out_hbm, ib0, ib1, rb0, rb1, gs0, gs1, ss0, ss1,
          is0, is1):
        wid = lax.axis_index("s") * NC + lax.axis_index("c")
        bb = wid * bn
        _gather_pipelined(h_hbm, out_hbm, idx2d, ib0, ib1, bb, bn, rb0, rb1,
                          gs0, gs1, ss0, ss1, is0, is1)

    return k


@functools.lru_cache(maxsize=None)
def _sc_scatter(E, N):
    bn = E // _EB_SC // NW
    assert bn % 2 == 0
    rt = (N // NS) & ~7
    rem = N - NS * rt
    mesh = plsc.VectorSubcoreMesh(core_axis_name="c", subcore_axis_name="s",
                                  num_cores=NC, num_subcores=NS)
    out_type = jax.ShapeDtypeStruct((NC, N, 128), jnp.float32)
    scratch = [pltpu.VMEM((bn, _EB_SC), jnp.int32),
               pltpu.VMEM((_EB_SC, 128), jnp.float32),
               pltpu.VMEM((_EB_SC, 128), jnp.float32),
               pltpu.VMEM_SHARED((N, 128), jnp.float32)] + \
              [pltpu.SemaphoreType.DMA] * 4

    @functools.partial(pl.kernel, out_type=out_type, mesh=mesh,
                       scratch_types=scratch)
    def k(rows_hbm, dst2d, zeros_hbm, out_hbm, ibuf, rb0, rb1, acc, ls0,
          ls1, cs0, cs1):
        sid = lax.axis_index("s")
        cid = lax.axis_index("c")
        wid = sid * NC + cid
        bb = wid * bn
        r0 = sid * rt
        pltpu.sync_copy(zeros_hbm.at[pl.ds(r0, rt)], acc.at[pl.ds(r0, rt)])
        if rem:
            @pl.when(sid == NS - 1)
            def _():
                pltpu.sync_copy(zeros_hbm.at[pl.ds(NS * rt, rem)],
                                acc.at[pl.ds(NS * rt, rem)])
        pltpu.sync_copy(dst2d.at[pl.ds(bb, bn)], ibuf)
        plsc.subcore_barrier()

        def l_start(j, rb, sem):
            pltpu.async_copy(rows_hbm.at[pl.ds((bb + j) * _EB_SC, _EB_SC)],
                             rb, sem)

        def l_wait(rb, sem):
            pltpu.make_async_copy(rows_hbm.at[pl.ds(0, _EB_SC)], rb,
                                  sem).wait()

        def c_start(j, rb, sem):
            pltpu.async_copy(rb, acc.at[ibuf.at[j]], sem, add=True)

        def c_wait(rb, sem):
            pltpu.make_async_copy(rb, acc.at[ibuf.at[0]], sem).wait()

        l_start(0, rb0, ls0)

        def pair(p, _):
            j0 = 2 * p
            j1 = j0 + 1
            l_start(j1, rb1, ls1)
            l_wait(rb0, ls0)
            c_start(j0, rb0, cs0)
            l_wait(rb1, ls1)
            c_wait(rb0, cs0)
            c_start(j1, rb1, cs1)

            @pl.when(j0 + 2 < bn)
            def _():
                l_start(j0 + 2, rb0, ls0)

            c_wait(rb1, cs1)
            return 0

        lax.fori_loop(0, bn // 2, pair, 0)
        plsc.subcore_barrier()
        pltpu.sync_copy(acc.at[pl.ds(r0, rt)],
                        out_hbm.at[cid, pl.ds(r0, rt)])
        if rem:
            @pl.when(sid == NS - 1)
            def _():
                pltpu.sync_copy(acc.at[pl.ds(NS * rt, rem)],
                                out_hbm.at[cid, pl.ds(NS * rt, rem)])

    return k


_RB = 2000
_EB = 512


def _tc_embed(x, W, b):
    n, _ = x.shape
    h = W.shape[1]

    def body(x_ref, w_ref, b_ref, o_ref, o16_ref):
        v = _leaky(
            jnp.dot(x_ref[...], w_ref[...],
                    preferred_element_type=jnp.float32) + b_ref[...])
        o_ref[...] = v
        o16_ref[...] = _pack_in_kernel(v)

    return pl.pallas_call(
        body,
        grid=(n // _RB,),
        in_specs=[
            pl.BlockSpec((_RB, x.shape[1]), lambda i: (i, 0)),
            pl.BlockSpec(W.shape, lambda i: (0, 0)),
            pl.BlockSpec((1, h), lambda i: (0, 0)),
        ],
        out_specs=[pl.BlockSpec((_RB, h), lambda i: (i, 0)),
                   pl.BlockSpec((_RB, h // 2), lambda i: (i, 0))],
        out_shape=[jax.ShapeDtypeStruct((n, h), jnp.float32),
                   jax.ShapeDtypeStruct((n, h // 2), jnp.float32)],
    )(x, W, b.reshape(1, h))


def _tc_edge_mlp(hs_parts, hd_parts, w_in, W1_parts, b1, W2, b2):
    E = w_in.shape[0]
    P = len(hs_parts)
    hid = b1.shape[0]
    nchunks = sum(2 * p.shape[1] // 128 for p in hs_parts)

    def body(*refs):
        hs = refs[0:P]
        hd = refs[P:2 * P]
        w_ref = refs[2 * P]
        W1 = refs[2 * P + 1:3 * P + 1]
        b1_ref, W2_ref, b2_ref = refs[3 * P + 1:3 * P + 4]
        w_out_ref = refs[3 * P + 4]
        r_refs = refs[3 * P + 5:]
        m = b1_ref[...]
        hs_un = []
        for p in range(P):
            alo, ahi = _unpack_halves(hs[p][...])
            blo, bhi = _unpack_halves(hd[p][...])
            hs_un.append((alo, ahi))
            d = jnp.concatenate(
                [jnp.abs(alo - blo), jnp.abs(ahi - bhi)], axis=1)
            m = m + jnp.dot(d, W1[p][...], preferred_element_type=jnp.float32)
        m = jnp.maximum(m, 0.0)
        s = jax.nn.sigmoid(
            jnp.dot(m, W2_ref[...], preferred_element_type=jnp.float32)
            + b2_ref[...])
        w = w_ref[...] * s[:, 0]
        w_out_ref[...] = w
        ci = 0
        for p in range(P):
            for half in hs_un[p]:
                r_refs[ci][...] = w[:, None] * half
                ci += 1

    in_specs = (
        [pl.BlockSpec((_EB, p.shape[1]), lambda i: (i, 0)) for p in hs_parts]
        + [pl.BlockSpec((_EB, p.shape[1]), lambda i: (i, 0)) for p in hd_parts]
        + [pl.BlockSpec((_EB,), lambda i: (i,))]
        + [pl.BlockSpec(Wp.shape, lambda i: (0, 0)) for Wp in W1_parts]
        + [pl.BlockSpec((1, hid), lambda i: (0, 0)),
           pl.BlockSpec(W2.shape, lambda i: (0, 0)),
           pl.BlockSpec((1, 1), lambda i: (0, 0))]
    )
    out_specs = (
        [pl.BlockSpec((_EB,), lambda i: (i,))]
        + [pl.BlockSpec((_EB, 128), lambda i: (i, 0))] * nchunks
    )
    out_shape = (
        [jax.ShapeDtypeStruct((E,), jnp.float32)]
        + [jax.ShapeDtypeStruct((E, 128), jnp.float32)] * nchunks
    )
    res = pl.pallas_call(
        body,
        grid=(E // _EB,),
        in_specs=in_specs,
        out_specs=out_specs,
        out_shape=out_shape,
    )(*hs_parts, *hd_parts, w_in, *W1_parts, b1.reshape(1, hid), W2,
      b2.reshape(1, 1))
    return res[0], list(res[1:])


def _tc_dense_bn(h_parts, Wt_parts, agg_chunks, Wb_chunks, b):
    n = h_parts[0].shape[0]
    hid = b.shape[0]
    P = len(h_parts)
    C = len(agg_chunks)
    nblk = n // _RB

    def body(*refs):
        hp = refs[0:P]
        Wt = refs[P:2 * P]
        ag = refs[2 * P:2 * P + C]
        Wb = refs[2 * P + C:2 * P + 2 * C]
        b_ref = refs[2 * P + 2 * C]
        y_ref, mu_ref, var_ref = refs[2 * P + 2 * C + 1:2 * P + 2 * C + 4]
        s_ref, q_ref = refs[2 * P + 2 * C + 4:]
        i = pl.program_id(0)
        y = b_ref[...]
        for p in range(P):
            y = y + jnp.dot(hp[p][...], Wt[p][...],
                            preferred_element_type=jnp.float32)
        for c in range(C):
            a = ag[c][0] + ag[c][1]
            y = y + jnp.dot(a, Wb[c][...], preferred_element_type=jnp.float32)
        y_ref[...] = y

        @pl.when(i == 0)
        def _():
            s_ref[...] = jnp.zeros_like(s_ref)
            q_ref[...] = jnp.zeros_like(q_ref)

        s_ref[...] += jnp.sum(y, axis=0, keepdims=True)
        q_ref[...] += jnp.sum(y * y, axis=0, keepdims=True)

        @pl.when(i == nblk - 1)
        def _():
            mu = s_ref[...] / n
            mu_ref[...] = mu
            var_ref[...] = q_ref[...] / n - mu * mu

    in_specs = (
        [pl.BlockSpec((_RB, p.shape[1]), lambda i: (i, 0)) for p in h_parts]
        + [pl.BlockSpec(Wp.shape, lambda i: (0, 0)) for Wp in Wt_parts]
        + [pl.BlockSpec((NC, _RB, 128), lambda i: (0, i, 0))
           for _ in agg_chunks]
        + [pl.BlockSpec(Wc.shape, lambda i: (0, 0)) for Wc in Wb_chunks]
        + [pl.BlockSpec((1, hid), lambda i: (0, 0))]
    )
    out_specs = [pl.BlockSpec((_RB, hid), lambda i: (i, 0)),
                 pl.BlockSpec((1, hid), lambda i: (0, 0)),
                 pl.BlockSpec((1, hid), lambda i: (0, 0))]
    out_shape = [jax.ShapeDtypeStruct((n, hid), jnp.float32),
                 jax.ShapeDtypeStruct((1, hid), jnp.float32),
                 jax.ShapeDtypeStruct((1, hid), jnp.float32)]
    y, mu, var = pl.pallas_call(
        body,
        grid=(nblk,),
        in_specs=in_specs,
        out_specs=out_specs,
        out_shape=out_shape,
        scratch_shapes=[pltpu.VMEM((1, hid), jnp.float32),
                        pltpu.VMEM((1, hid), jnp.float32)],
    )(*h_parts, *Wt_parts, *agg_chunks, *Wb_chunks, b.reshape(1, hid))
    return y, mu, var


def _tc_bn_apply(y, mu, var, gamma, beta):
    n, hid = y.shape

    def body(y_ref, mu_ref, var_ref, g_ref, be_ref, o_ref, o16_ref):
        inv = g_ref[...] * lax.rsqrt(var_ref[...] + 1e-5)
        v = _leaky((y_ref[...] - mu_ref[...]) * inv + be_ref[...])
        o_ref[...] = v
        o16_ref[...] = _pack_in_kernel(v)

    return pl.pallas_call(
        body,
        grid=(n // _RB,),
        in_specs=[pl.BlockSpec((_RB, hid), lambda i: (i, 0)),
                  pl.BlockSpec((1, hid), lambda i: (0, 0)),
                  pl.BlockSpec((1, hid), lambda i: (0, 0)),
                  pl.BlockSpec((1, hid), lambda i: (0, 0)),
                  pl.BlockSpec((1, hid), lambda i: (0, 0))],
        out_specs=[pl.BlockSpec((_RB, hid), lambda i: (i, 0)),
                   pl.BlockSpec((_RB, hid // 2), lambda i: (i, 0))],
        out_shape=[jax.ShapeDtypeStruct((n, hid), jnp.float32),
                   jax.ShapeDtypeStruct((n, hid // 2), jnp.float32)],
    )(y, mu, var, gamma.reshape(1, hid), beta.reshape(1, hid))


def _tc_matmul_parts(h_parts, W_parts, out_w, extra=None,
                     out_dtype=jnp.float32):
    n = h_parts[0].shape[0]
    P = len(h_parts)
    has_extra = extra is not None

    def body(*refs):
        hp = refs[0:P]
        Wp = refs[P:2 * P]
        k = 2 * P
        acc = jnp.zeros((_RB, out_w), jnp.float32)
        if has_extra:
            acc = refs[k][0] + refs[k][1]
            k += 1
        o_ref = refs[k]
        for p in range(P):
            acc = acc + jnp.dot(hp[p][...], Wp[p][...],
                                preferred_element_type=jnp.float32)
        o_ref[...] = acc.astype(out_dtype)

    in_specs = (
        [pl.BlockSpec((_RB, p.shape[1]), lambda i: (i, 0)) for p in h_parts]
        + [pl.BlockSpec(Wp.shape, lambda i: (0, 0)) for Wp in W_parts]
    )
    args = list(h_parts) + list(W_parts)
    if has_extra:
        in_specs.append(pl.BlockSpec((NC, _RB, out_w), lambda i: (0, i, 0)))
        args.append(extra)
    return pl.pallas_call(
        body,
        grid=(n // _RB,),
        in_specs=in_specs,
        out_specs=pl.BlockSpec((_RB, out_w), lambda i: (i, 0)),
        out_shape=jax.ShapeDtypeStruct((n, out_w), out_dtype),
    )(*args)


def _tc_add_partials(base, extra):
    n, w = base.shape

    def body(b_ref, e_ref, o_ref):
        o_ref[...] = b_ref[...] + e_ref[0] + e_ref[1]

    return pl.pallas_call(
        body,
        grid=(n // _RB,),
        in_specs=[pl.BlockSpec((_RB, w), lambda i: (i, 0)),
                  pl.BlockSpec((NC, _RB, w), lambda i: (0, i, 0))],
        out_specs=pl.BlockSpec((_RB, w), lambda i: (i, 0)),
        out_shape=jax.ShapeDtypeStruct((n, w), jnp.float32),
    )(base, extra)


def _tc_scale_rows(rows, w):
    E, width = rows.shape

    def body(r_ref, w_ref, o_ref):
        o_ref[...] = w_ref[...][:, None] * r_ref[...].astype(jnp.float32)

    return pl.pallas_call(
        body,
        grid=(E // _EB,),
        in_specs=[pl.BlockSpec((_EB, width), lambda i: (i, 0)),
                  pl.BlockSpec((_EB,), lambda i: (i,))],
        out_specs=pl.BlockSpec((_EB, width), lambda i: (i, 0)),
        out_shape=jax.ShapeDtypeStruct((E, width), jnp.float32),
    )(rows, w)


def kernel(x, edge_index, edge_weight, g_size, emb_W, emb_b, wc1_W1, wc1_b1,
           wc1_W2, wc1_b2, gc1_W, gc1_b, gc1_g, gc1_be, wc2_W1, wc2_b1,
           wc2_W2, wc2_b2, gc2_W, gc2_b, gc2_g, gc2_be, gcl_W):
    N, _ = x.shape
    E = edge_index.shape[1]
    HID = emb_W.shape[1]
    chunk = NW * _EB_SC * 2
    EP = ((E + chunk - 1) // chunk) * chunk
    pad_idx = (jnp.arange(EP - E, dtype=jnp.int32) * 8) % N
    src = jnp.concatenate([edge_index[0], pad_idx]).reshape(
        EP // _EB_SC, _EB_SC)
    dst = jnp.concatenate([edge_index[1], pad_idx]).reshape(
        EP // _EB_SC, _EB_SC)
    ew = jnp.pad(edge_weight, (0, EP - E))
    zeros128 = jnp.zeros((N, 128), jnp.float32)

    h0, h0b = _tc_embed(x, emb_W, emb_b)

    g2p = _sc_gather2(EP, N, HID // 2)
    hs0, hd0 = g2p(h0b, src, dst)
    w1, r1 = _tc_edge_mlp([hs0], [hd0], ew, [wc1_W1], wc1_b1,
                          wc1_W2, wc1_b2)
    aggs1 = [_sc_scatter(EP, N)(r, dst, zeros128) for r in r1]
    y1, mu1, var1 = _tc_dense_bn(
        [h0], [gc1_W[:HID]], aggs1,
        [gc1_W[HID + 128 * c:HID + 128 * (c + 1)] for c in range(2)], gc1_b)
    hn1, hn1b = _tc_bn_apply(y1, mu1, var1, gc1_g, gc1_be)

    hsn, hdn = g2p(hn1b, src, dst)
    w2, r2 = _tc_edge_mlp([hs0, hsn], [hd0, hdn], w1,
                          [wc2_W1[:HID], wc2_W1[HID:]],
                          wc2_b1, wc2_W2, wc2_b2)
    aggs2 = [_sc_scatter(EP, N)(r, dst, zeros128) for r in r2]
    y2, mu2, var2 = _tc_dense_bn(
        [h0, hn1], [gc2_W[:HID], gc2_W[HID:2 * HID]], aggs2,
        [gc2_W[2 * HID + 128 * c:2 * HID + 128 * (c + 1)] for c in range(4)],
        gc2_b)
    hn2, _hn2b = _tc_bn_apply(y2, mu2, var2, gc2_g, gc2_be)

    OUT = gcl_W.shape[1]
    h2_parts = [h0, hn1, hn2]
    Wb = [gcl_W[3 * HID + HID * p:3 * HID + HID * (p + 1)] for p in range(3)]
    z = _tc_matmul_parts(h2_parts, Wb, OUT)
    zs = _sc_gather1(EP, N, OUT)(z, src)
    rfin = _tc_scale_rows(zs, w2)
    aggf = _sc_scatter(EP, N)(rfin, dst, zeros128)
    Wt = [gcl_W[HID * p:HID * (p + 1)] for p in range(3)]
    out_base = _tc_matmul_parts(h2_parts, Wt, OUT)
    out = _tc_add_partials(out_base, aggf)

    return (out, w2[:E], g_size)
